# baseline (speedup 1.0000x reference)
import functools
import math

import jax
import jax.numpy as jnp
import numpy as np
from jax import lax
from jax.experimental import pallas as pl
from jax.experimental.pallas import tpu as pltpu
from jax.experimental.pallas import tpu_sc as plsc

N0 = 10000
E0 = 320000
HF = 128
NL = 3
NSTAIN = 4
RATIO = 0.5

NTILES = 32
NP = 10112
RPT = NP // 16
ECHUNK = 128
NCHUNK = 96
EPT = ECHUNK * NCHUNK
RPD = NP // NTILES


def _build_sc_edge_kernel(with_deg):
    mesh = plsc.VectorSubcoreMesh(core_axis_name="c", subcore_axis_name="s")
    out_type = [jax.ShapeDtypeStruct((2, NP, HF), jnp.float32)]
    scratch = [
        pltpu.VMEM((NCHUNK, ECHUNK), jnp.int32),
        pltpu.VMEM((NCHUNK, ECHUNK), jnp.int32),
        pltpu.VMEM((ECHUNK, HF), jnp.float32),
        pltpu.SemaphoreType.DMA,
        pltpu.VMEM_SHARED((NP, HF), jnp.float32),
    ]
    if with_deg:
        out_type.append(jax.ShapeDtypeStruct((2 * NP,), jnp.float32))
        scratch += [
            pltpu.VMEM((ECHUNK,), jnp.float32),
            pltpu.VMEM_SHARED((NP,), jnp.float32),
            pltpu.VMEM((RPT,), jnp.float32),
        ]

    def body(*refs):
        if with_deg:
            (h_hbm, src_hbm, dst_hbm, alive_hbm,
             out_feat, out_deg,
             src_v, dst_v, rows_v, sem, acc_sh, vals_v, deg_sh,
             dout_v) = refs
        else:
            (h_hbm, src_hbm, dst_hbm,
             out_feat,
             src_v, dst_v, rows_v, sem, acc_sh) = refs

        cid = lax.axis_index("c")
        sid = lax.axis_index("s")
        wid = sid * 2 + cid
        zrow = jnp.zeros((16,), jnp.float32)

        def zbuf(i, _):
            for u in range(HF // 16):
                rows_v[i, pl.ds(u * 16, 16)] = zrow
            return 0
        lax.fori_loop(0, ECHUNK, zbuf, 0)
        base_r = sid * RPT
        for off in range(0, RPT, ECHUNK):
            nrows = min(ECHUNK, RPT - off)
            pltpu.sync_copy(rows_v.at[pl.ds(0, nrows)],
                            acc_sh.at[pl.ds(base_r + off, nrows)])
        if with_deg:
            for off in (0, 128, 256, 384, 504):
                pltpu.sync_copy(rows_v.at[0],
                                deg_sh.at[pl.ds(base_r + off, ECHUNK)])

        pltpu.sync_copy(src_hbm.at[wid], src_v)
        pltpu.sync_copy(dst_hbm.at[wid], dst_v)

        plsc.subcore_barrier()

        def step(j, _):
            pltpu.async_copy(h_hbm.at[src_v.at[j]], rows_v, sem).wait()
            pltpu.sync_copy(rows_v, acc_sh.at[dst_v.at[j]], add=True)
            if with_deg:
                pltpu.async_copy(alive_hbm.at[src_v.at[j]], vals_v, sem).wait()
                pltpu.sync_copy(vals_v, deg_sh.at[dst_v.at[j]], add=True)
            return 0
        lax.fori_loop(0, NCHUNK, step, 0)

        plsc.subcore_barrier()

        pltpu.sync_copy(acc_sh.at[pl.ds(base_r, RPT)],
                        out_feat.at[cid, pl.ds(base_r, RPT)])
        if with_deg:
            pltpu.sync_copy(deg_sh.at[pl.ds(base_r, RPT)], dout_v)
            pltpu.sync_copy(dout_v, out_deg.at[pl.ds(cid * NP + base_r, RPT)])

    return functools.partial(
        pl.kernel, mesh=mesh, out_type=out_type, scratch_types=scratch,
        compiler_params=pltpu.CompilerParams(needs_layout_passes=False))(body)


_sc_edge_feat_deg = _build_sc_edge_kernel(True)
_sc_edge_feat = _build_sc_edge_kernel(False)


NBLK = 8
NB = NP // NBLK


def _tc_conv_body(aggp_ref, degt_ref, h_ref, alive_ref, wl_ref, bl_ref,
                  wr_ref, hn_ref):
    agg = aggp_ref[0, :, :] + aggp_ref[1, :, :]
    deg = degt_ref[:, 0:1] + degt_ref[:, 1:2]
    mean_nb = agg / jnp.maximum(deg, 1.0)
    pre = (jax.lax.dot(mean_nb, wl_ref[...])
           + bl_ref[...]
           + jax.lax.dot(h_ref[...], wr_ref[...]))
    hn_ref[...] = jnp.maximum(pre, 0.0) * alive_ref[...]


def _tc_conv(aggp, degt, h, alive2d, wl, bl, wr):
    return pl.pallas_call(
        _tc_conv_body,
        grid=(NBLK,),
        in_specs=[
            pl.BlockSpec((2, NB, HF), lambda i: (0, i, 0)),
            pl.BlockSpec((NB, 2), lambda i: (i, 0)),
            pl.BlockSpec((NB, HF), lambda i: (i, 0)),
            pl.BlockSpec((NB, 1), lambda i: (i, 0)),
            pl.BlockSpec((HF, HF), lambda i: (0, 0)),
            pl.BlockSpec((1, HF), lambda i: (0, 0)),
            pl.BlockSpec((HF, HF), lambda i: (0, 0)),
        ],
        out_specs=pl.BlockSpec((NB, HF), lambda i: (i, 0)),
        out_shape=jax.ShapeDtypeStruct((NP, HF), jnp.float32),
    )(aggp, degt, h, alive2d, wl, bl, wr)


def _tc_score_body(aggp_ref, hn_ref, alive_ref, wrel_ref, brel_ref,
                   wroot_ref, score_ref):
    aggp = aggp_ref[0, :, :] + aggp_ref[1, :, :]
    pre = (jax.lax.dot(aggp, wrel_ref[...])
           + brel_ref[0, 0]
           + jax.lax.dot(hn_ref[...], wroot_ref[...]))
    score = jnp.tanh(pre)
    score_ref[...] = jnp.where(alive_ref[...] > 0.0, score, -2.0)


def _tc_score(aggp, hn, alive2d, wrel, brel, wroot):
    return pl.pallas_call(
        _tc_score_body,
        grid=(NBLK,),
        in_specs=[
            pl.BlockSpec((2, NB, HF), lambda i: (0, i, 0)),
            pl.BlockSpec((NB, HF), lambda i: (i, 0)),
            pl.BlockSpec((NB, 1), lambda i: (i, 0)),
            pl.BlockSpec((HF, 1), lambda i: (0, 0)),
            pl.BlockSpec((1, 1), lambda i: (0, 0)),
            pl.BlockSpec((HF, 1), lambda i: (0, 0)),
        ],
        out_specs=pl.BlockSpec((NB, 1), lambda i: (i, 0)),
        out_shape=jax.ShapeDtypeStruct((NP, 1), jnp.float32),
    )(aggp, hn, alive2d, wrel, brel, wroot)


def _monotone_u32(score):
    b = jax.lax.bitcast_convert_type(score, jnp.uint32)
    return jnp.where((b >> 31) == 0, b ^ jnp.uint32(0x80000000), ~b)


def _tc_pool_body(k, score_ref, hn_ref, onehot_ref, hng_ref, alive_new_ref,
                  emb_ref, cnt_s, sg_s, fsum_s, fmax_s):
    i = pl.program_id(0)
    dn01 = (((0,), (0,)), ((), ()))
    big = jnp.float32(3.0e38)

    u_full = _monotone_u32(score_ref[...])

    def bit_step(j, t):
        cand = t | (jnp.uint32(1) << (jnp.uint32(31) - j.astype(jnp.uint32)))
        c = jnp.sum((u_full >= cand).astype(jnp.int32))
        return jnp.where(c >= k, cand, t)
    thresh = lax.fori_loop(0, 32, bit_step, jnp.uint32(0))

    cgt = jnp.sum((u_full > thresh).astype(jnp.int32))
    r = k - cgt
    idx_full = jax.lax.broadcasted_iota(jnp.int32, (NP, 1), 0)
    tie_full = u_full == thresh

    def ibit_step(j, t):
        cand = t | jnp.left_shift(jnp.int32(1), jnp.int32(13) - j)
        c = jnp.sum((tie_full & (idx_full < cand)).astype(jnp.int32))
        return jnp.where(c <= r, cand, t)
    tidx = lax.fori_loop(0, 14, ibit_step, jnp.int32(0))

    score = score_ref[pl.ds(i * NB, NB), :]
    ub = _monotone_u32(score)
    idx_blk = jax.lax.broadcasted_iota(jnp.int32, (NB, 1), 0) + i * NB
    sel = (ub > thresh) | ((ub == thresh) & (idx_blk < tidx))
    g = jnp.where(sel, score, 0.0)
    hng = hn_ref[...] * g
    hng_ref[...] = hng
    alive_new = sel.astype(jnp.float32)
    alive_new_ref[...] = alive_new

    @pl.when(i == 0)
    def _init():
        cnt_s[...] = jnp.zeros_like(cnt_s)
        sg_s[...] = jnp.zeros_like(sg_s)
        fsum_s[...] = jnp.zeros_like(fsum_s)
        fmax_s[...] = jnp.full_like(fmax_s, -big)

    onehot = onehot_ref[...]
    msk = onehot * alive_new
    cnt_s[...] += jax.lax.dot_general(onehot, alive_new, dn01)
    sg_s[...] += jax.lax.dot_general(onehot, g, dn01)
    fsum_s[...] += jax.lax.dot_general(msk, hng, dn01)
    for s in range(NSTAIN):
        m = msk[:, s:s + 1] > 0.0
        fx = jnp.max(jnp.where(m, hng, -big), axis=0, keepdims=True)
        fmax_s[s:s + 1, :] = jnp.maximum(fmax_s[s:s + 1, :], fx)

    @pl.when(i == NBLK - 1)
    def _finalize():
        cnt = cnt_s[...]
        presf = (cnt > 0.0).astype(jnp.float32)
        ms = presf * sg_s[...] / jnp.maximum(cnt, 1.0)
        w = jnp.where(presf > 0.0, ms / jnp.sum(ms), 0.0)
        fmean = fsum_s[...] / jnp.maximum(cnt, 1.0)
        wmean = jax.lax.dot_general(w, fmean, dn01)
        fmax = jnp.where(presf > 0.0, fmax_s[...], 0.0)
        wmax = jax.lax.dot_general(w, fmax, dn01)
        emb_ref[...] = jnp.concatenate([wmean, wmax], axis=1)


def _tc_pool(k, score, hn, onehot):
    return pl.pallas_call(
        functools.partial(_tc_pool_body, k),
        grid=(NBLK,),
        in_specs=[
            pl.BlockSpec((NP, 1), lambda i: (0, 0)),
            pl.BlockSpec((NB, HF), lambda i: (i, 0)),
            pl.BlockSpec((NB, NSTAIN), lambda i: (i, 0)),
        ],
        out_specs=[
            pl.BlockSpec((NB, HF), lambda i: (i, 0)),
            pl.BlockSpec((NB, 1), lambda i: (i, 0)),
            pl.BlockSpec((1, 2 * HF), lambda i: (0, 0)),
        ],
        out_shape=[
            jax.ShapeDtypeStruct((NP, HF), jnp.float32),
            jax.ShapeDtypeStruct((NP, 1), jnp.float32),
            jax.ShapeDtypeStruct((1, 2 * HF), jnp.float32),
        ],
        scratch_shapes=[
            pltpu.VMEM((NSTAIN, 1), jnp.float32),
            pltpu.VMEM((NSTAIN, 1), jnp.float32),
            pltpu.VMEM((NSTAIN, HF), jnp.float32),
            pltpu.VMEM((NSTAIN, HF), jnp.float32),
        ],
    )(score, hn, onehot)


def _layernorm(x, g, b):
    m = jnp.mean(x, axis=-1, keepdims=True)
    v = jnp.mean((x - m) ** 2, axis=-1, keepdims=True)
    return (x - m) / jnp.sqrt(v + 1e-5) * g + b


def _tc_head_body(xcat_ref, ln1g_ref, ln1b_ref, wv_ref, bv_ref, wo_ref, bo_ref,
                  ln2g_ref, ln2b_ref, lamask_ref, wc1_ref, bc1_ref, wc2_ref,
                  bc2_ref, logits_ref, probs_ref, la_ref):
    dnt = (((1,), (1,)), ((), ()))
    xn = _layernorm(xcat_ref[...], ln1g_ref[...], ln1b_ref[...])
    v = jax.lax.dot_general(xn, wv_ref[...], dnt) + bv_ref[...]
    attn = jax.lax.dot_general(v, wo_ref[...], dnt) + bo_ref[...]
    y = _layernorm(attn + xn, ln2g_ref[...], ln2b_ref[...])
    la = jax.lax.dot(y, lamask_ref[...])
    la = la - jnp.min(la)
    la = la + 1e-8
    la = la / jnp.sum(la)
    la_ref[...] = la
    z = jnp.maximum(y, 0.0)
    h1 = jax.lax.dot_general(z, wc1_ref[...], dnt) + bc1_ref[...]
    logits = jax.lax.dot_general(h1, wc2_ref[...], dnt) + bc2_ref[...]
    logits_ref[...] = logits
    e = jnp.exp(logits - jnp.max(logits, axis=-1, keepdims=True))
    probs_ref[...] = e / jnp.sum(e, axis=-1, keepdims=True)


def _tc_head(xcat, ln1g, ln1b, wv, bv, wo, bo, ln2g, ln2b, lamask, wc1, bc1,
             wc2, bc2):
    return pl.pallas_call(
        _tc_head_body,
        out_shape=[
            jax.ShapeDtypeStruct((1, 2), jnp.float32),
            jax.ShapeDtypeStruct((1, 2), jnp.float32),
            jax.ShapeDtypeStruct((1, NL), jnp.float32),
        ],
    )(xcat, ln1g, ln1b, wv, bv, wo, bo, ln2g, ln2b, lamask, wc1, bc1, wc2, bc2)


def kernel(x, edge_index, node_attr, batch, label, sage_Wl, sage_bl, sage_Wr,
           pool_Wrel, pool_brel, pool_Wroot, ln1_g, ln1_b, Wqkv, bqkv, Wo, bo,
           ln2_g, ln2_b, Wc1, bc1, Wc2, bc2):
    del batch
    h = jnp.zeros((NP, HF), jnp.float32).at[:N0].set(x)
    src = edge_index[0].astype(jnp.int32)
    dst = edge_index[1].astype(jnp.int32)
    b = dst // RPD
    order = jnp.argsort(b, stable=True)
    bs = b[order]
    cnt = jnp.zeros((NTILES,), jnp.int32).at[b].add(1)
    off = jnp.concatenate([jnp.zeros((1,), jnp.int32),
                           jnp.cumsum(cnt)[:-1].astype(jnp.int32)])
    rank = jnp.arange(E0, dtype=jnp.int32) - off[bs]
    slot = bs * EPT + rank
    src3d = jnp.full((NTILES * EPT,), N0, jnp.int32).at[slot].set(
        src[order], mode="drop").reshape(NTILES, NCHUNK, ECHUNK)
    dst3d = jnp.broadcast_to(
        (jnp.arange(NTILES, dtype=jnp.int32) * RPD)[:, None], (NTILES, EPT)
    ).reshape(-1).at[slot].set(dst[order], mode="drop"
                               ).reshape(NTILES, NCHUNK, ECHUNK)
    alive1d = jnp.zeros((NP,), jnp.float32).at[:N0].set(1.0)
    alive = alive1d[:, None]
    na = jnp.full((NP,), -1, jnp.int32).at[:N0].set(node_attr.astype(jnp.int32))
    onehot = (na[:, None] == jnp.arange(NSTAIN, dtype=jnp.int32)[None, :]
              ).astype(jnp.float32)

    embs = []
    n = N0
    for i in range(NL):
        aggp, degp = _sc_edge_feat_deg(h, src3d, dst3d, alive1d)
        degt = jnp.transpose(degp.reshape(2, NP))
        hn = _tc_conv(aggp, degt, h, alive, sage_Wl[i],
                      sage_bl[i][None, :], sage_Wr[i])
        aggp2, = _sc_edge_feat(hn, src3d, dst3d)
        score = _tc_score(aggp2, hn, alive, pool_Wrel[i],
                          pool_brel[i][None, :], pool_Wroot[i])
        k = int(math.ceil(RATIO * n))
        h, alive, emb = _tc_pool(k, score, hn, onehot)
        alive1d = alive[:, 0]
        embs.append(emb)
        n = k

    xcat = jnp.concatenate(embs, axis=1)
    D = 2 * HF * NL
    Wv = Wqkv[2 * D:3 * D]
    bv = bqkv[2 * D:3 * D][None, :]
    lamask = np.zeros((D, NL), np.float32)
    for i in range(NL):
        lamask[i * 2 * HF:(i + 1) * 2 * HF, i] = 1.0
    logits, probs, la = _tc_head(
        xcat, ln1_g[None, :], ln1_b[None, :], Wv, bv, Wo, bo[None, :],
        ln2_g[None, :], ln2_b[None, :], jnp.asarray(lamask), Wc1, bc1[None, :],
        Wc2, bc2[None, :])
    return (logits, probs, la.reshape(NL), label)
